# baseline (speedup 1.0000x reference)
import functools
import jax
import jax.numpy as jnp
from jax import lax
from jax.experimental import pallas as pl
from jax.experimental.pallas import tpu as pltpu

_USE_SC = True

if _USE_SC:
    from jax.experimental.pallas import tpu_sc as plsc

BN = 1024


def _sc_gather(emb, rflat, uidx, iidx, ni):
    if not _USE_SC:
        u = jnp.take(emb, uidx, axis=0)
        rq = jnp.take(rflat, uidx * ni + iidx)
        return u, rq
    n, d = emb.shape
    b = uidx.shape[0]
    info = plsc.get_sparse_core_info()
    nc = info.num_cores
    nw = nc * info.num_subcores
    bpw = b // nw
    mesh = plsc.VectorSubcoreMesh(core_axis_name="c", subcore_axis_name="s")

    @functools.partial(
        pl.kernel, mesh=mesh,
        out_type=(jax.ShapeDtypeStruct((b, d), jnp.float32),
                  jax.ShapeDtypeStruct((b,), jnp.float32)),
        scratch_types=[
            pltpu.VMEM((bpw,), jnp.int32),
            pltpu.VMEM((bpw,), jnp.int32),
            pltpu.VMEM((bpw,), jnp.int32),
            pltpu.VMEM((bpw, d), jnp.float32),
            pltpu.VMEM((bpw,), jnp.float32),
            pltpu.SemaphoreType.DMA,
            pltpu.SemaphoreType.DMA,
        ])
    def k(emb_hbm, rflat_hbm, uidx_hbm, iidx_hbm, u_out, rq_out,
          uvec, ivec, fvec, rows, rqv, sem_u, sem_q):
        wid = lax.axis_index("s") * nc + lax.axis_index("c")
        base = wid * bpw
        pltpu.sync_copy(uidx_hbm.at[pl.ds(base, bpw)], uvec)
        pltpu.sync_copy(iidx_hbm.at[pl.ds(base, bpw)], ivec)
        cu = pltpu.async_copy(emb_hbm.at[uvec], rows, sem_u)
        for t in range(bpw // 16):
            s = pl.ds(t * 16, 16)
            fvec[s] = uvec[s] * ni + ivec[s]
        cq = pltpu.async_copy(rflat_hbm.at[fvec], rqv, sem_q)
        cu.wait()
        pltpu.sync_copy(rows, u_out.at[pl.ds(base, bpw)])
        cq.wait()
        pltpu.sync_copy(rqv, rq_out.at[pl.ds(base, bpw)])

    return k(emb, rflat, uidx, iidx)


def _sc_epilogue(avg_flat, tnum, tden, rq, uidx):
    if not _USE_SC:
        avgu = jnp.take(avg_flat, uidx)
        vq = rq == rq
        rq0 = jnp.where(vq, rq, 0.0)
        num = tnum - jnp.where(vq, rq0 - avgu, 0.0)
        den = tden - jnp.where(vq, 1.0, 0.0)
        safe = jnp.where(den == 0.0, 1.0, den)
        return jnp.where(den == 0.0, avgu, avgu + num / safe)
    b = uidx.shape[0]
    info = plsc.get_sparse_core_info()
    nc = info.num_cores
    nw = nc * info.num_subcores
    bpw = b // nw
    mesh = plsc.VectorSubcoreMesh(core_axis_name="c", subcore_axis_name="s")

    @functools.partial(
        pl.kernel, mesh=mesh,
        out_type=jax.ShapeDtypeStruct((b,), jnp.float32),
        scratch_types=[
            pltpu.VMEM((bpw,), jnp.int32),
            pltpu.VMEM((bpw,), jnp.float32),
            pltpu.VMEM((bpw,), jnp.float32),
            pltpu.VMEM((bpw,), jnp.float32),
            pltpu.VMEM((bpw,), jnp.float32),
            pltpu.VMEM((bpw,), jnp.float32),
            pltpu.SemaphoreType.DMA,
        ])
    def ek(avg_hbm, tnum_hbm, tden_hbm, rq_hbm, uidx_hbm, pred_out,
           uvec, av, nv, dv, rv, pv, sem):
        wid = lax.axis_index("s") * nc + lax.axis_index("c")
        base = wid * bpw
        sl = pl.ds(base, bpw)
        pltpu.sync_copy(uidx_hbm.at[sl], uvec)
        ca = pltpu.async_copy(avg_hbm.at[uvec], av, sem)
        pltpu.sync_copy(tnum_hbm.at[sl], nv)
        pltpu.sync_copy(tden_hbm.at[sl], dv)
        pltpu.sync_copy(rq_hbm.at[sl], rv)
        ca.wait()
        for t in range(bpw // 16):
            s = pl.ds(t * 16, 16)
            rqv = rv[s]
            avgu = av[s]
            vq = rqv == rqv
            num = nv[s] - jnp.where(vq, rqv - avgu, 0.0)
            den = dv[s] - jnp.where(vq, 1.0, 0.0)
            safe = jnp.where(den == 0.0, 1.0, den)
            pv[s] = jnp.where(den == 0.0, avgu, avgu + num / safe)
        pltpu.sync_copy(pv, pred_out.at[sl])

    return ek(avg_flat, tnum, tden, rq, uidx)


def _main_body(iidx_ref, u_ref, r_ref, e_ref,
               tnum_ref, tden_ref, avg_ref,
               pden_ref, vt_ref, uhat_ref):
    i = pl.program_id(0)
    nb = pl.num_programs(0)
    bn, ni = r_ref.shape
    b, d = u_ref.shape

    @pl.when(i == 0)
    def _():
        u = u_ref[...]
        nu2 = jnp.sum(u * u, axis=1)
        uhat_ref[...] = (u * lax.rsqrt(jnp.maximum(nu2, 1e-60))[:, None]
                         ).astype(jnp.bfloat16)
        pden_ref[...] = jnp.zeros_like(pden_ref)
        vt_ref[...] = jnp.zeros_like(vt_ref)

    r = r_ref[...]
    validb = r == r
    validf = validb.astype(jnp.float32)
    valid_bf = validf.astype(jnp.bfloat16)
    r0 = jnp.where(validb, r, 0.0)
    ssum = jnp.sum(r0, axis=1)
    ones8 = jnp.ones((ni, 8), jnp.bfloat16)
    cnt = jax.lax.dot_general(
        valid_bf, ones8,
        dimension_numbers=(((1,), (0,)), ((), ())),
        preferred_element_type=jnp.float32)[:, 0]
    avg = jnp.where(cnt > 0.0, ssum / jnp.maximum(cnt, 1.0), 0.0)
    avg_ref[...] = avg[None, None, :]

    e = e_ref[...]
    nn2 = jnp.sum(e * e, axis=1)
    ehat = (e * lax.rsqrt(jnp.maximum(nn2, 1e-60))[:, None]
            ).astype(jnp.bfloat16)

    sim = jax.lax.dot_general(
        uhat_ref[...], ehat,
        dimension_numbers=(((1,), (1,)), ((), ())),
        preferred_element_type=jnp.float32).astype(jnp.bfloat16)

    pden_ref[...] += jax.lax.dot_general(
        jnp.abs(sim), valid_bf,
        dimension_numbers=(((1,), (0,)), ((), ())),
        preferred_element_type=jnp.float32)

    acomb = (r0 - avg[:, None] * validf).astype(jnp.bfloat16)
    vt_ref[...] += jax.lax.dot_general(
        acomb, ehat,
        dimension_numbers=(((0,), (0,)), ((), ())),
        preferred_element_type=jnp.float32)

    @pl.when(i == nb - 1)
    def _():
        j_ids = jax.lax.broadcasted_iota(jnp.int32, (b, ni), 1)
        sel = iidx_ref[...] == j_ids
        tden = jnp.sum(jnp.where(sel, pden_ref[...], 0.0), axis=1)
        vg = jax.lax.dot_general(
            sel.astype(jnp.float32), vt_ref[...],
            dimension_numbers=(((1,), (0,)), ((), ())),
            preferred_element_type=jnp.float32)
        tnum = jnp.sum(uhat_ref[...].astype(jnp.float32) * vg, axis=1)
        tnum_ref[...] = tnum[:, None]
        tden_ref[...] = tden[:, None]


def kernel(rating_matrix, user_embeddings, user_indices, item_indices):
    n_users, n_items = rating_matrix.shape
    d = user_embeddings.shape[1]
    b = user_indices.shape[0]
    nb = n_users // BN

    u, rq = _sc_gather(user_embeddings, rating_matrix.reshape(-1),
                       user_indices, item_indices, n_items)

    iidx_col = item_indices.reshape(b, 1)

    tnum, tden, avg_all = pl.pallas_call(
        _main_body,
        grid=(nb,),
        in_specs=[
            pl.BlockSpec((b, 1), lambda i: (0, 0)),
            pl.BlockSpec((b, d), lambda i: (0, 0)),
            pl.BlockSpec((BN, n_items), lambda i: (i, 0)),
            pl.BlockSpec((BN, d), lambda i: (i, 0)),
        ],
        out_specs=[
            pl.BlockSpec((b, 1), lambda i: (0, 0)),
            pl.BlockSpec((b, 1), lambda i: (0, 0)),
            pl.BlockSpec((1, 1, BN), lambda i: (i, 0, 0)),
        ],
        out_shape=[
            jax.ShapeDtypeStruct((b, 1), jnp.float32),
            jax.ShapeDtypeStruct((b, 1), jnp.float32),
            jax.ShapeDtypeStruct((nb, 1, BN), jnp.float32),
        ],
        scratch_shapes=[
            pltpu.VMEM((b, n_items), jnp.float32),
            pltpu.VMEM((n_items, d), jnp.float32),
            pltpu.VMEM((b, d), jnp.bfloat16),
        ],
    )(iidx_col, u, rating_matrix, user_embeddings)

    pred = _sc_epilogue(avg_all.reshape(n_users), tnum.reshape(b),
                        tden.reshape(b), rq, user_indices)
    return pred


# speedup vs baseline: 1.4011x; 1.4011x over previous
import functools
import jax
import jax.numpy as jnp
from jax import lax
from jax.experimental import pallas as pl
from jax.experimental.pallas import tpu as pltpu
from jax.experimental.pallas import tpu_sc as plsc

BN = 1024


def _sc_gather(emb, rmat, uidx):
    n, d = emb.shape
    ni = rmat.shape[1]
    b = uidx.shape[0]
    info = plsc.get_sparse_core_info()
    nc = info.num_cores
    nw = nc * info.num_subcores
    bpw = b // nw
    mesh = plsc.VectorSubcoreMesh(core_axis_name="c", subcore_axis_name="s")

    @functools.partial(
        pl.kernel, mesh=mesh,
        out_type=(jax.ShapeDtypeStruct((b, d), jnp.float32),
                  jax.ShapeDtypeStruct((b, ni), jnp.float32)),
        scratch_types=[
            pltpu.VMEM((bpw,), jnp.int32),
            pltpu.VMEM((bpw, d), jnp.float32),
            pltpu.VMEM((bpw, ni), jnp.float32),
            pltpu.SemaphoreType.DMA,
            pltpu.SemaphoreType.DMA,
        ])
    def k(emb_hbm, rmat_hbm, uidx_hbm, u_out, rr_out,
          uvec, rows, rrows, sem_u, sem_q):
        wid = lax.axis_index("s") * nc + lax.axis_index("c")
        base = wid * bpw
        pltpu.sync_copy(uidx_hbm.at[pl.ds(base, bpw)], uvec)
        cu = pltpu.async_copy(emb_hbm.at[uvec], rows, sem_u)
        cq = pltpu.async_copy(rmat_hbm.at[uvec], rrows, sem_q)
        cu.wait()
        pltpu.sync_copy(rows, u_out.at[pl.ds(base, bpw)])
        cq.wait()
        pltpu.sync_copy(rrows, rr_out.at[pl.ds(base, bpw)])

    return k(emb, rmat, uidx)


def _sc_epilogue(avg_flat, tnum, tden, rq, uidx):
    b = uidx.shape[0]
    info = plsc.get_sparse_core_info()
    nc = info.num_cores
    nw = nc * info.num_subcores
    bpw = b // nw
    mesh = plsc.VectorSubcoreMesh(core_axis_name="c", subcore_axis_name="s")

    @functools.partial(
        pl.kernel, mesh=mesh,
        out_type=jax.ShapeDtypeStruct((b,), jnp.float32),
        scratch_types=[
            pltpu.VMEM((bpw,), jnp.int32),
            pltpu.VMEM((bpw,), jnp.float32),
            pltpu.VMEM((bpw,), jnp.float32),
            pltpu.VMEM((bpw,), jnp.float32),
            pltpu.VMEM((bpw,), jnp.float32),
            pltpu.VMEM((bpw,), jnp.float32),
            pltpu.SemaphoreType.DMA,
        ])
    def ek(avg_hbm, tnum_hbm, tden_hbm, rq_hbm, uidx_hbm, pred_out,
           uvec, av, nv, dv, rv, pv, sem):
        wid = lax.axis_index("s") * nc + lax.axis_index("c")
        base = wid * bpw
        sl = pl.ds(base, bpw)
        pltpu.sync_copy(uidx_hbm.at[sl], uvec)
        ca = pltpu.async_copy(avg_hbm.at[uvec], av, sem)
        pltpu.sync_copy(tnum_hbm.at[sl], nv)
        pltpu.sync_copy(tden_hbm.at[sl], dv)
        pltpu.sync_copy(rq_hbm.at[sl], rv)
        ca.wait()
        for t in range(bpw // 16):
            s = pl.ds(t * 16, 16)
            rqv = rv[s]
            avgu = av[s]
            vq = rqv == rqv
            num = nv[s] - jnp.where(vq, rqv - avgu, 0.0)
            den = dv[s] - jnp.where(vq, 1.0, 0.0)
            safe = jnp.where(den == 0.0, 1.0, den)
            pv[s] = jnp.where(den == 0.0, avgu, avgu + num / safe)
        pltpu.sync_copy(pv, pred_out.at[sl])

    return ek(avg_flat, tnum, tden, rq, uidx)


def _main_body(iidx_ref, u_ref, rr_ref, r_ref, e_ref,
               tnum_ref, tden_ref, rq_ref, avg_ref,
               pden_ref, vt_ref, uhat_ref):
    i = pl.program_id(0)
    nb = pl.num_programs(0)
    bn, ni = r_ref.shape
    b, d = u_ref.shape

    @pl.when(i == 0)
    def _():
        u = u_ref[...]
        nu2 = jnp.sum(u * u, axis=1)
        uhat_ref[...] = (u * lax.rsqrt(jnp.maximum(nu2, 1e-60))[:, None]
                         ).astype(jnp.bfloat16)
        pden_ref[...] = jnp.zeros_like(pden_ref)
        vt_ref[...] = jnp.zeros_like(vt_ref)

    r = r_ref[...]
    validb = r == r
    valid_bf = validb.astype(jnp.bfloat16)
    r0 = jnp.where(validb, r, 0.0)
    ssum = jnp.sum(r0, axis=1)
    ones8 = jnp.ones((ni, 8), jnp.bfloat16)
    cnt = jax.lax.dot_general(
        valid_bf, ones8,
        dimension_numbers=(((1,), (0,)), ((), ())),
        preferred_element_type=jnp.float32)[:, 0]
    avg = jnp.where(cnt > 0.0, ssum / jnp.maximum(cnt, 1.0), 0.0)
    avg_ref[...] = avg[None, None, :]

    e = e_ref[...]
    nn2 = jnp.sum(e * e, axis=1)
    ehat = (e * lax.rsqrt(jnp.maximum(nn2, 1e-60))[:, None]
            ).astype(jnp.bfloat16)

    sim = jax.lax.dot_general(
        uhat_ref[...], ehat,
        dimension_numbers=(((1,), (1,)), ((), ())),
        preferred_element_type=jnp.float32).astype(jnp.bfloat16)

    pden_ref[...] += jax.lax.dot_general(
        jnp.abs(sim), valid_bf,
        dimension_numbers=(((1,), (0,)), ((), ())),
        preferred_element_type=jnp.float32)

    acomb = jnp.where(validb, r - avg[:, None], 0.0).astype(jnp.bfloat16)
    vt_ref[...] += jax.lax.dot_general(
        acomb, ehat,
        dimension_numbers=(((0,), (0,)), ((), ())),
        preferred_element_type=jnp.float32)

    @pl.when(i == nb - 1)
    def _():
        j_ids = jax.lax.broadcasted_iota(jnp.int32, (b, ni), 1)
        sel = iidx_ref[...] == j_ids
        tden = jnp.sum(jnp.where(sel, pden_ref[...], 0.0), axis=1)
        rq = jnp.sum(jnp.where(sel, rr_ref[...], 0.0), axis=1)
        vg = jax.lax.dot_general(
            sel.astype(jnp.float32), vt_ref[...],
            dimension_numbers=(((1,), (0,)), ((), ())),
            preferred_element_type=jnp.float32)
        tnum = jnp.sum(uhat_ref[...].astype(jnp.float32) * vg, axis=1)
        tnum_ref[...] = tnum[:, None]
        tden_ref[...] = tden[:, None]
        rq_ref[...] = rq[:, None]


def kernel(rating_matrix, user_embeddings, user_indices, item_indices):
    n_users, n_items = rating_matrix.shape
    d = user_embeddings.shape[1]
    b = user_indices.shape[0]
    nb = n_users // BN

    u, rq_rows = _sc_gather(user_embeddings, rating_matrix, user_indices)

    iidx_col = item_indices.reshape(b, 1)

    tnum, tden, rq, avg_all = pl.pallas_call(
        _main_body,
        grid=(nb,),
        in_specs=[
            pl.BlockSpec((b, 1), lambda i: (0, 0)),
            pl.BlockSpec((b, d), lambda i: (0, 0)),
            pl.BlockSpec((b, n_items), lambda i: (0, 0)),
            pl.BlockSpec((BN, n_items), lambda i: (i, 0)),
            pl.BlockSpec((BN, d), lambda i: (i, 0)),
        ],
        out_specs=[
            pl.BlockSpec((b, 1), lambda i: (0, 0)),
            pl.BlockSpec((b, 1), lambda i: (0, 0)),
            pl.BlockSpec((b, 1), lambda i: (0, 0)),
            pl.BlockSpec((1, 1, BN), lambda i: (i, 0, 0)),
        ],
        out_shape=[
            jax.ShapeDtypeStruct((b, 1), jnp.float32),
            jax.ShapeDtypeStruct((b, 1), jnp.float32),
            jax.ShapeDtypeStruct((b, 1), jnp.float32),
            jax.ShapeDtypeStruct((nb, 1, BN), jnp.float32),
        ],
        scratch_shapes=[
            pltpu.VMEM((b, n_items), jnp.float32),
            pltpu.VMEM((n_items, d), jnp.float32),
            pltpu.VMEM((b, d), jnp.bfloat16),
        ],
    )(iidx_col, u, rq_rows, rating_matrix, user_embeddings)

    pred = _sc_epilogue(avg_all.reshape(n_users), tnum.reshape(b),
                        tden.reshape(b), rq.reshape(b), user_indices)
    return pred


# speedup vs baseline: 1.5529x; 1.1084x over previous
import functools
import jax
import jax.numpy as jnp
from jax import lax
from jax.experimental import pallas as pl
from jax.experimental.pallas import tpu as pltpu
from jax.experimental.pallas import tpu_sc as plsc

BN = 2048


def _sc_gather(emb, rmat, uidx):
    n, d = emb.shape
    ni = rmat.shape[1]
    b = uidx.shape[0]
    info = plsc.get_sparse_core_info()
    nc = info.num_cores
    nw = nc * info.num_subcores
    bpw = b // nw
    mesh = plsc.VectorSubcoreMesh(core_axis_name="c", subcore_axis_name="s")

    @functools.partial(
        pl.kernel, mesh=mesh,
        out_type=(jax.ShapeDtypeStruct((b, d), jnp.float32),
                  jax.ShapeDtypeStruct((b, ni), jnp.float32)),
        scratch_types=[
            pltpu.VMEM((bpw,), jnp.int32),
            pltpu.VMEM((bpw, d), jnp.float32),
            pltpu.VMEM((bpw, ni), jnp.float32),
            pltpu.SemaphoreType.DMA,
            pltpu.SemaphoreType.DMA,
        ])
    def k(emb_hbm, rmat_hbm, uidx_hbm, u_out, rr_out,
          uvec, rows, rrows, sem_u, sem_q):
        wid = lax.axis_index("s") * nc + lax.axis_index("c")
        base = wid * bpw
        pltpu.sync_copy(uidx_hbm.at[pl.ds(base, bpw)], uvec)
        cu = pltpu.async_copy(emb_hbm.at[uvec], rows, sem_u)
        cq = pltpu.async_copy(rmat_hbm.at[uvec], rrows, sem_q)
        cu.wait()
        pltpu.sync_copy(rows, u_out.at[pl.ds(base, bpw)])
        cq.wait()
        pltpu.sync_copy(rrows, rr_out.at[pl.ds(base, bpw)])

    return k(emb, rmat, uidx)


def _sc_epilogue(avg_flat, tnum, tden, rq, uidx):
    b = uidx.shape[0]
    info = plsc.get_sparse_core_info()
    nc = info.num_cores
    nw = nc * info.num_subcores
    bpw = b // nw
    mesh = plsc.VectorSubcoreMesh(core_axis_name="c", subcore_axis_name="s")

    @functools.partial(
        pl.kernel, mesh=mesh,
        out_type=jax.ShapeDtypeStruct((b,), jnp.float32),
        scratch_types=[
            pltpu.VMEM((bpw,), jnp.int32),
            pltpu.VMEM((bpw,), jnp.float32),
            pltpu.VMEM((bpw,), jnp.float32),
            pltpu.VMEM((bpw,), jnp.float32),
            pltpu.VMEM((bpw,), jnp.float32),
            pltpu.VMEM((bpw,), jnp.float32),
            pltpu.SemaphoreType.DMA,
        ])
    def ek(avg_hbm, tnum_hbm, tden_hbm, rq_hbm, uidx_hbm, pred_out,
           uvec, av, nv, dv, rv, pv, sem):
        wid = lax.axis_index("s") * nc + lax.axis_index("c")
        base = wid * bpw
        sl = pl.ds(base, bpw)
        pltpu.sync_copy(uidx_hbm.at[sl], uvec)
        ca = pltpu.async_copy(avg_hbm.at[uvec], av, sem)
        pltpu.sync_copy(tnum_hbm.at[sl], nv)
        pltpu.sync_copy(tden_hbm.at[sl], dv)
        pltpu.sync_copy(rq_hbm.at[sl], rv)
        ca.wait()
        for t in range(bpw // 16):
            s = pl.ds(t * 16, 16)
            rqv = rv[s]
            avgu = av[s]
            vq = rqv == rqv
            num = nv[s] - jnp.where(vq, rqv - avgu, 0.0)
            den = dv[s] - jnp.where(vq, 1.0, 0.0)
            safe = jnp.where(den == 0.0, 1.0, den)
            pv[s] = jnp.where(den == 0.0, avgu, avgu + num / safe)
        pltpu.sync_copy(pv, pred_out.at[sl])

    return ek(avg_flat, tnum, tden, rq, uidx)


def _main_body(iidx_ref, u_ref, rr_ref, r_ref, e_ref,
               tnum_ref, tden_ref, rq_ref, avg_ref,
               pden_ref, vt_ref, uhat_ref):
    i = pl.program_id(0)
    nb = pl.num_programs(0)
    bn, ni = r_ref.shape
    b, d = u_ref.shape

    @pl.when(i == 0)
    def _():
        u = u_ref[...]
        nu2 = jnp.sum(u * u, axis=1)
        uhat_ref[...] = (u * lax.rsqrt(jnp.maximum(nu2, 1e-60))[:, None]
                         ).astype(jnp.bfloat16)
        pden_ref[...] = jnp.zeros_like(pden_ref)
        vt_ref[...] = jnp.zeros_like(vt_ref)

    r = r_ref[...]
    validb = r == r
    valid_bf = validb.astype(jnp.bfloat16)
    r0 = jnp.where(validb, r, 0.0)
    ssum = jnp.sum(r0, axis=1)
    ones8 = jnp.ones((ni, 8), jnp.bfloat16)
    cnt = jax.lax.dot_general(
        valid_bf, ones8,
        dimension_numbers=(((1,), (0,)), ((), ())),
        preferred_element_type=jnp.float32)[:, 0]
    avg = jnp.where(cnt > 0.0, ssum / jnp.maximum(cnt, 1.0), 0.0)
    avg_ref[...] = avg[None, None, :]

    e = e_ref[...]
    nn2 = jnp.sum(e * e, axis=1)
    ehat = (e * lax.rsqrt(jnp.maximum(nn2, 1e-60))[:, None]
            ).astype(jnp.bfloat16)

    sim = jax.lax.dot_general(
        uhat_ref[...], ehat,
        dimension_numbers=(((1,), (1,)), ((), ())),
        preferred_element_type=jnp.float32).astype(jnp.bfloat16)

    pden_ref[...] += jax.lax.dot_general(
        jnp.abs(sim).astype(jnp.float8_e4m3fn),
        validb.astype(jnp.float8_e4m3fn),
        dimension_numbers=(((1,), (0,)), ((), ())),
        preferred_element_type=jnp.float32)

    acomb = jnp.where(validb, r - avg[:, None], 0.0).astype(jnp.bfloat16)
    vt_ref[...] += jax.lax.dot_general(
        acomb, ehat,
        dimension_numbers=(((0,), (0,)), ((), ())),
        preferred_element_type=jnp.float32)

    @pl.when(i == nb - 1)
    def _():
        j_ids = jax.lax.broadcasted_iota(jnp.int32, (b, ni), 1)
        sel = iidx_ref[...] == j_ids
        tden = jnp.sum(jnp.where(sel, pden_ref[...], 0.0), axis=1)
        rq = jnp.sum(jnp.where(sel, rr_ref[...], 0.0), axis=1)
        vg = jax.lax.dot_general(
            sel.astype(jnp.float32), vt_ref[...],
            dimension_numbers=(((1,), (0,)), ((), ())),
            preferred_element_type=jnp.float32)
        tnum = jnp.sum(uhat_ref[...].astype(jnp.float32) * vg, axis=1)
        tnum_ref[...] = tnum[:, None]
        tden_ref[...] = tden[:, None]
        rq_ref[...] = rq[:, None]


def kernel(rating_matrix, user_embeddings, user_indices, item_indices):
    n_users, n_items = rating_matrix.shape
    d = user_embeddings.shape[1]
    b = user_indices.shape[0]
    nb = n_users // BN

    u, rq_rows = _sc_gather(user_embeddings, rating_matrix, user_indices)

    iidx_col = item_indices.reshape(b, 1)

    tnum, tden, rq, avg_all = pl.pallas_call(
        _main_body,
        grid=(nb,),
        in_specs=[
            pl.BlockSpec((b, 1), lambda i: (0, 0)),
            pl.BlockSpec((b, d), lambda i: (0, 0)),
            pl.BlockSpec((b, n_items), lambda i: (0, 0)),
            pl.BlockSpec((BN, n_items), lambda i: (i, 0)),
            pl.BlockSpec((BN, d), lambda i: (i, 0)),
        ],
        out_specs=[
            pl.BlockSpec((b, 1), lambda i: (0, 0)),
            pl.BlockSpec((b, 1), lambda i: (0, 0)),
            pl.BlockSpec((b, 1), lambda i: (0, 0)),
            pl.BlockSpec((1, 1, BN), lambda i: (i, 0, 0)),
        ],
        out_shape=[
            jax.ShapeDtypeStruct((b, 1), jnp.float32),
            jax.ShapeDtypeStruct((b, 1), jnp.float32),
            jax.ShapeDtypeStruct((b, 1), jnp.float32),
            jax.ShapeDtypeStruct((nb, 1, BN), jnp.float32),
        ],
        scratch_shapes=[
            pltpu.VMEM((b, n_items), jnp.float32),
            pltpu.VMEM((n_items, d), jnp.float32),
            pltpu.VMEM((b, d), jnp.bfloat16),
        ],
    )(iidx_col, u, rq_rows, rating_matrix, user_embeddings)

    pred = _sc_epilogue(avg_all.reshape(n_users), tnum.reshape(b),
                        tden.reshape(b), rq.reshape(b), user_indices)
    return pred


# speedup vs baseline: 1.6251x; 1.0465x over previous
import functools
import jax
import jax.numpy as jnp
from jax import lax
from jax.experimental import pallas as pl
from jax.experimental.pallas import tpu as pltpu
from jax.experimental.pallas import tpu_sc as plsc

BN = 2048


def _sc_gather(emb, rmat, uidx):
    n, d = emb.shape
    ni = rmat.shape[1]
    b = uidx.shape[0]
    info = plsc.get_sparse_core_info()
    nc = info.num_cores
    nw = nc * info.num_subcores
    bpw = b // nw
    mesh = plsc.VectorSubcoreMesh(core_axis_name="c", subcore_axis_name="s")

    @functools.partial(
        pl.kernel, mesh=mesh,
        out_type=(jax.ShapeDtypeStruct((b, d), jnp.float32),
                  jax.ShapeDtypeStruct((b, ni), jnp.float32)),
        scratch_types=[
            pltpu.VMEM((bpw,), jnp.int32),
            pltpu.VMEM((bpw, d), jnp.float32),
            pltpu.VMEM((bpw, ni), jnp.float32),
            pltpu.SemaphoreType.DMA,
            pltpu.SemaphoreType.DMA,
        ])
    def k(emb_hbm, rmat_hbm, uidx_hbm, u_out, rr_out,
          uvec, rows, rrows, sem_u, sem_q):
        wid = lax.axis_index("s") * nc + lax.axis_index("c")
        base = wid * bpw
        pltpu.sync_copy(uidx_hbm.at[pl.ds(base, bpw)], uvec)
        cu = pltpu.async_copy(emb_hbm.at[uvec], rows, sem_u)
        cq = pltpu.async_copy(rmat_hbm.at[uvec], rrows, sem_q)
        cu.wait()
        pltpu.sync_copy(rows, u_out.at[pl.ds(base, bpw)])
        cq.wait()
        pltpu.sync_copy(rrows, rr_out.at[pl.ds(base, bpw)])

    return k(emb, rmat, uidx)


def _sc_epilogue(avg_flat, tnum, tden, rq, uidx):
    b = uidx.shape[0]
    info = plsc.get_sparse_core_info()
    nc = info.num_cores
    nw = nc * info.num_subcores
    bpw = b // nw
    mesh = plsc.VectorSubcoreMesh(core_axis_name="c", subcore_axis_name="s")

    @functools.partial(
        pl.kernel, mesh=mesh,
        out_type=jax.ShapeDtypeStruct((b,), jnp.float32),
        scratch_types=[
            pltpu.VMEM((bpw,), jnp.int32),
            pltpu.VMEM((bpw,), jnp.float32),
            pltpu.VMEM((bpw,), jnp.float32),
            pltpu.VMEM((bpw,), jnp.float32),
            pltpu.VMEM((bpw,), jnp.float32),
            pltpu.VMEM((bpw,), jnp.float32),
            pltpu.SemaphoreType.DMA,
        ])
    def ek(avg_hbm, tnum_hbm, tden_hbm, rq_hbm, uidx_hbm, pred_out,
           uvec, av, nv, dv, rv, pv, sem):
        wid = lax.axis_index("s") * nc + lax.axis_index("c")
        base = wid * bpw
        sl = pl.ds(base, bpw)
        pltpu.sync_copy(uidx_hbm.at[sl], uvec)
        ca = pltpu.async_copy(avg_hbm.at[uvec], av, sem)
        pltpu.sync_copy(tnum_hbm.at[sl], nv)
        pltpu.sync_copy(tden_hbm.at[sl], dv)
        pltpu.sync_copy(rq_hbm.at[sl], rv)
        ca.wait()
        for t in range(bpw // 16):
            s = pl.ds(t * 16, 16)
            rqv = rv[s]
            avgu = av[s]
            vq = rqv == rqv
            num = nv[s] - jnp.where(vq, rqv - avgu, 0.0)
            den = dv[s] - jnp.where(vq, 1.0, 0.0)
            safe = jnp.where(den == 0.0, 1.0, den)
            pv[s] = jnp.where(den == 0.0, avgu, avgu + num / safe)
        pltpu.sync_copy(pv, pred_out.at[sl])

    return ek(avg_flat, tnum, tden, rq, uidx)


def _main_body(iidx_ref, u_ref, rr_ref, r_ref, e_ref,
               tnum_ref, tden_ref, rq_ref, avg_ref,
               pden_ref, vt_ref, uhat_ref):
    i = pl.program_id(0)
    nb = pl.num_programs(0)
    bn, ni = r_ref.shape
    b, d = u_ref.shape

    @pl.when(i == 0)
    def _():
        u = u_ref[...]
        nu2 = jnp.sum(u * u, axis=1)
        uhat_ref[...] = (u * lax.rsqrt(jnp.maximum(nu2, 1e-60))[:, None]
                         ).astype(jnp.bfloat16)
        pden_ref[...] = jnp.zeros_like(pden_ref)
        vt_ref[...] = jnp.zeros_like(vt_ref)

    r = r_ref[...]
    validb = r == r
    valid_f8 = validb.astype(jnp.float8_e4m3fn)
    r0 = jnp.where(validb, r, 0.0)
    ssum = jnp.sum(r0, axis=1)
    ones8 = jnp.ones((ni, 8), jnp.float8_e4m3fn)
    cnt = jax.lax.dot_general(
        valid_f8, ones8,
        dimension_numbers=(((1,), (0,)), ((), ())),
        preferred_element_type=jnp.float32)[:, 0]
    avg = jnp.where(cnt > 0.0, ssum / jnp.maximum(cnt, 1.0), 0.0)
    avg_ref[...] = avg[None, None, :]

    e = e_ref[...]
    nn2 = jnp.sum(e * e, axis=1)
    ehat = (e * lax.rsqrt(jnp.maximum(nn2, 1e-60))[:, None]
            ).astype(jnp.bfloat16)

    sim = jax.lax.dot_general(
        uhat_ref[...], ehat,
        dimension_numbers=(((1,), (1,)), ((), ())),
        preferred_element_type=jnp.float32).astype(jnp.bfloat16)

    pden_ref[...] += jax.lax.dot_general(
        jnp.abs(sim).astype(jnp.float8_e4m3fn),
        valid_f8,
        dimension_numbers=(((1,), (0,)), ((), ())),
        preferred_element_type=jnp.float32)

    acomb = jnp.where(validb, r - avg[:, None], 0.0).astype(jnp.bfloat16)
    vt_ref[...] += jax.lax.dot_general(
        ehat, acomb,
        dimension_numbers=(((0,), (0,)), ((), ())),
        preferred_element_type=jnp.float32)

    @pl.when(i == nb - 1)
    def _():
        j_ids = jax.lax.broadcasted_iota(jnp.int32, (b, ni), 1)
        sel = iidx_ref[...] == j_ids
        tden = jnp.sum(jnp.where(sel, pden_ref[...], 0.0), axis=1)
        rq = jnp.sum(jnp.where(sel, rr_ref[...], 0.0), axis=1)
        tnum_full = jax.lax.dot_general(
            uhat_ref[...].astype(jnp.float32), vt_ref[...],
            dimension_numbers=(((1,), (0,)), ((), ())),
            preferred_element_type=jnp.float32)
        tnum = jnp.sum(jnp.where(sel, tnum_full, 0.0), axis=1)
        tnum_ref[...] = tnum[:, None]
        tden_ref[...] = tden[:, None]
        rq_ref[...] = rq[:, None]


def kernel(rating_matrix, user_embeddings, user_indices, item_indices):
    n_users, n_items = rating_matrix.shape
    d = user_embeddings.shape[1]
    b = user_indices.shape[0]
    nb = n_users // BN

    u, rq_rows = _sc_gather(user_embeddings, rating_matrix, user_indices)

    iidx_col = item_indices.reshape(b, 1)

    tnum, tden, rq, avg_all = pl.pallas_call(
        _main_body,
        grid=(nb,),
        in_specs=[
            pl.BlockSpec((b, 1), lambda i: (0, 0)),
            pl.BlockSpec((b, d), lambda i: (0, 0)),
            pl.BlockSpec((b, n_items), lambda i: (0, 0)),
            pl.BlockSpec((BN, n_items), lambda i: (i, 0)),
            pl.BlockSpec((BN, d), lambda i: (i, 0)),
        ],
        out_specs=[
            pl.BlockSpec((b, 1), lambda i: (0, 0)),
            pl.BlockSpec((b, 1), lambda i: (0, 0)),
            pl.BlockSpec((b, 1), lambda i: (0, 0)),
            pl.BlockSpec((1, 1, BN), lambda i: (i, 0, 0)),
        ],
        out_shape=[
            jax.ShapeDtypeStruct((b, 1), jnp.float32),
            jax.ShapeDtypeStruct((b, 1), jnp.float32),
            jax.ShapeDtypeStruct((b, 1), jnp.float32),
            jax.ShapeDtypeStruct((nb, 1, BN), jnp.float32),
        ],
        scratch_shapes=[
            pltpu.VMEM((b, n_items), jnp.float32),
            pltpu.VMEM((d, n_items), jnp.float32),
            pltpu.VMEM((b, d), jnp.bfloat16),
        ],
    )(iidx_col, u, rq_rows, rating_matrix, user_embeddings)

    pred = _sc_epilogue(avg_all.reshape(n_users), tnum.reshape(b),
                        tden.reshape(b), rq.reshape(b), user_indices)
    return pred


# speedup vs baseline: 1.6784x; 1.0328x over previous
import functools
import jax
import jax.numpy as jnp
from jax import lax
from jax.experimental import pallas as pl
from jax.experimental.pallas import tpu as pltpu
from jax.experimental.pallas import tpu_sc as plsc

BN = 2048


def _sc_gather(emb, rmat, uidx):
    n, d = emb.shape
    ni = rmat.shape[1]
    b = uidx.shape[0]
    info = plsc.get_sparse_core_info()
    nc = info.num_cores
    nw = nc * info.num_subcores
    bpw = b // nw
    mesh = plsc.VectorSubcoreMesh(core_axis_name="c", subcore_axis_name="s")

    @functools.partial(
        pl.kernel, mesh=mesh,
        out_type=(jax.ShapeDtypeStruct((b, d), jnp.float32),
                  jax.ShapeDtypeStruct((b, ni), jnp.float32)),
        scratch_types=[
            pltpu.VMEM((bpw,), jnp.int32),
            pltpu.VMEM((bpw, d), jnp.float32),
            pltpu.VMEM((bpw, ni), jnp.float32),
            pltpu.SemaphoreType.DMA,
            pltpu.SemaphoreType.DMA,
        ])
    def k(emb_hbm, rmat_hbm, uidx_hbm, u_out, rr_out,
          uvec, rows, rrows, sem_u, sem_q):
        wid = lax.axis_index("s") * nc + lax.axis_index("c")
        base = wid * bpw
        pltpu.sync_copy(uidx_hbm.at[pl.ds(base, bpw)], uvec)
        cu = pltpu.async_copy(emb_hbm.at[uvec], rows, sem_u)
        cq = pltpu.async_copy(rmat_hbm.at[uvec], rrows, sem_q)
        cu.wait()
        pltpu.sync_copy(rows, u_out.at[pl.ds(base, bpw)])
        cq.wait()
        pltpu.sync_copy(rrows, rr_out.at[pl.ds(base, bpw)])

    return k(emb, rmat, uidx)


def _sc_epilogue(avg_flat, tnum, tden, rq, uidx):
    b = uidx.shape[0]
    info = plsc.get_sparse_core_info()
    nc = info.num_cores
    nw = nc * info.num_subcores
    bpw = b // nw
    mesh = plsc.VectorSubcoreMesh(core_axis_name="c", subcore_axis_name="s")

    @functools.partial(
        pl.kernel, mesh=mesh,
        out_type=jax.ShapeDtypeStruct((b,), jnp.float32),
        scratch_types=[
            pltpu.VMEM((bpw,), jnp.int32),
            pltpu.VMEM((bpw,), jnp.float32),
            pltpu.VMEM((bpw,), jnp.float32),
            pltpu.VMEM((bpw,), jnp.float32),
            pltpu.VMEM((bpw,), jnp.float32),
            pltpu.VMEM((bpw,), jnp.float32),
            pltpu.SemaphoreType.DMA,
        ])
    def ek(avg_hbm, tnum_hbm, tden_hbm, rq_hbm, uidx_hbm, pred_out,
           uvec, av, nv, dv, rv, pv, sem):
        wid = lax.axis_index("s") * nc + lax.axis_index("c")
        base = wid * bpw
        sl = pl.ds(base, bpw)
        pltpu.sync_copy(uidx_hbm.at[sl], uvec)
        ca = pltpu.async_copy(avg_hbm.at[uvec], av, sem)
        pltpu.sync_copy(tnum_hbm.at[sl], nv)
        pltpu.sync_copy(tden_hbm.at[sl], dv)
        pltpu.sync_copy(rq_hbm.at[sl], rv)
        ca.wait()
        for t in range(bpw // 16):
            s = pl.ds(t * 16, 16)
            rqv = rv[s]
            avgu = av[s]
            vq = rqv == rqv
            num = nv[s] - jnp.where(vq, rqv - avgu, 0.0)
            den = dv[s] - jnp.where(vq, 1.0, 0.0)
            safe = jnp.where(den == 0.0, 1.0, den)
            pv[s] = jnp.where(den == 0.0, avgu, avgu + num / safe)
        pltpu.sync_copy(pv, pred_out.at[sl])

    return ek(avg_flat, tnum, tden, rq, uidx)


def _main_body(iidx_ref, u_ref, rr_ref, r_ref, e_ref,
               tnum_ref, tden_ref, rq_ref, avg_ref,
               pden_ref, vt_ref, uhat_ref):
    i = pl.program_id(0)
    nb = pl.num_programs(0)
    bn, ni = r_ref.shape
    b, d = u_ref.shape

    @pl.when(i == 0)
    def _():
        u = u_ref[...]
        nu2 = jnp.sum(u * u, axis=1)
        uhat_ref[...] = (u * lax.rsqrt(jnp.maximum(nu2, 1e-60))[:, None]
                         ).astype(jnp.bfloat16)
        pden_ref[...] = jnp.zeros_like(pden_ref)
        vt_ref[...] = jnp.zeros_like(vt_ref)

    r = r_ref[...]
    validb = r == r
    valid_f8 = validb.astype(jnp.float8_e4m3fn)
    r0 = jnp.where(validb, r, 0.0)
    ssum = jnp.sum(r0, axis=1)
    ones8 = jnp.ones((ni, 8), jnp.float8_e4m3fn)
    cnt = jax.lax.dot_general(
        valid_f8, ones8,
        dimension_numbers=(((1,), (0,)), ((), ())),
        preferred_element_type=jnp.float32)[:, 0]
    avg = jnp.where(cnt > 0.0, ssum / jnp.maximum(cnt, 1.0), 0.0)
    avg_ref[...] = avg[None, None, :]

    e = e_ref[...]
    nn2 = jnp.sum(e * e, axis=1)
    ehat = (e * lax.rsqrt(jnp.maximum(nn2, 1e-60))[:, None]
            ).astype(jnp.bfloat16)

    sim = jax.lax.dot_general(
        uhat_ref[...].astype(jnp.float8_e4m3fn), ehat.astype(jnp.float8_e4m3fn),
        dimension_numbers=(((1,), (1,)), ((), ())),
        preferred_element_type=jnp.float32)

    pden_ref[...] += jax.lax.dot_general(
        jnp.abs(sim).astype(jnp.float8_e4m3fn),
        valid_f8,
        dimension_numbers=(((1,), (0,)), ((), ())),
        preferred_element_type=jnp.float32)

    acomb = jnp.where(validb, r - avg[:, None], 0.0).astype(jnp.bfloat16)
    vt_ref[...] += jax.lax.dot_general(
        ehat, acomb,
        dimension_numbers=(((0,), (0,)), ((), ())),
        preferred_element_type=jnp.float32)

    @pl.when(i == nb - 1)
    def _():
        j_ids = jax.lax.broadcasted_iota(jnp.int32, (b, ni), 1)
        sel = iidx_ref[...] == j_ids
        tden = jnp.sum(jnp.where(sel, pden_ref[...], 0.0), axis=1)
        rq = jnp.sum(jnp.where(sel, rr_ref[...], 0.0), axis=1)
        tnum_full = jax.lax.dot_general(
            uhat_ref[...].astype(jnp.float32), vt_ref[...],
            dimension_numbers=(((1,), (0,)), ((), ())),
            preferred_element_type=jnp.float32)
        tnum = jnp.sum(jnp.where(sel, tnum_full, 0.0), axis=1)
        tnum_ref[...] = tnum[:, None]
        tden_ref[...] = tden[:, None]
        rq_ref[...] = rq[:, None]


def kernel(rating_matrix, user_embeddings, user_indices, item_indices):
    n_users, n_items = rating_matrix.shape
    d = user_embeddings.shape[1]
    b = user_indices.shape[0]
    nb = n_users // BN

    u, rq_rows = _sc_gather(user_embeddings, rating_matrix, user_indices)

    iidx_col = item_indices.reshape(b, 1)

    tnum, tden, rq, avg_all = pl.pallas_call(
        _main_body,
        grid=(nb,),
        in_specs=[
            pl.BlockSpec((b, 1), lambda i: (0, 0)),
            pl.BlockSpec((b, d), lambda i: (0, 0)),
            pl.BlockSpec((b, n_items), lambda i: (0, 0)),
            pl.BlockSpec((BN, n_items), lambda i: (i, 0)),
            pl.BlockSpec((BN, d), lambda i: (i, 0)),
        ],
        out_specs=[
            pl.BlockSpec((b, 1), lambda i: (0, 0)),
            pl.BlockSpec((b, 1), lambda i: (0, 0)),
            pl.BlockSpec((b, 1), lambda i: (0, 0)),
            pl.BlockSpec((1, 1, BN), lambda i: (i, 0, 0)),
        ],
        out_shape=[
            jax.ShapeDtypeStruct((b, 1), jnp.float32),
            jax.ShapeDtypeStruct((b, 1), jnp.float32),
            jax.ShapeDtypeStruct((b, 1), jnp.float32),
            jax.ShapeDtypeStruct((nb, 1, BN), jnp.float32),
        ],
        scratch_shapes=[
            pltpu.VMEM((b, n_items), jnp.float32),
            pltpu.VMEM((d, n_items), jnp.float32),
            pltpu.VMEM((b, d), jnp.bfloat16),
        ],
    )(iidx_col, u, rq_rows, rating_matrix, user_embeddings)

    pred = _sc_epilogue(avg_all.reshape(n_users), tnum.reshape(b),
                        tden.reshape(b), rq.reshape(b), user_indices)
    return pred
